# baseline (speedup 1.0000x reference)
import functools

import jax
import jax.numpy as jnp
from jax import lax
from jax.experimental import pallas as pl
from jax.experimental.pallas import tpu as pltpu
from jax.experimental.pallas import tpu_sc as plsc

N = 10000
D = 256
E = 160000
DH = D // 2

NC = 2
NS = 16
NW = NC * NS
L = 16

CH = 80
NCHUNK = 128
IG = 8
NG = NCHUNK // IG
NG2 = NG // 2
NBUF = 4
EPS = NCHUNK * CH
E_PAD = NS * EPS
EPW = E_PAD // NW
NPT = N // NS
N_ACC = N + 8
N_DEG = 10016

_BN = 1000
_GRID = N // _BN

_mesh = plsc.VectorSubcoreMesh(
    core_axis_name="c", subcore_axis_name="s", num_cores=NC, num_subcores=NS
)

_sc_params = pltpu.CompilerParams(needs_layout_passes=False)


@functools.partial(
    pl.kernel,
    out_type=jax.ShapeDtypeStruct((NW, N), jnp.float32),
    mesh=_mesh,
    scratch_types=[
        pltpu.VMEM((EPW,), jnp.int32),
        pltpu.VMEM((N,), jnp.float32),
    ],
    compiler_params=_sc_params,
)
def _sc_deg(dst_hbm, out_hbm, dstv, degp):
    cid = lax.axis_index("c")
    sid = lax.axis_index("s")
    wid = cid * NS + sid

    pltpu.sync_copy(dst_hbm.at[wid], dstv)

    zeros = jnp.zeros((L,), jnp.float32)

    def zbody(i, _):
        degp[pl.ds(i * L, L)] = zeros
        return _

    lax.fori_loop(0, N // L, zbody, None)

    ones = jnp.ones((L,), jnp.float32)

    nvec = jnp.full((L,), N, jnp.int32)

    def cbody(j, _):
        idx = dstv[pl.ds(j * L, L)]
        plsc.addupdate_scatter(degp, [idx], ones, mask=idx < nvec)
        return _

    lax.fori_loop(0, EPW // L, cbody, None)

    pltpu.sync_copy(degp, out_hbm.at[wid])


@functools.partial(
    pl.kernel,
    out_type=jax.ShapeDtypeStruct((NW, NPT, DH), jnp.float32),
    mesh=_mesh,
    scratch_types=[
        pltpu.VMEM((2 * IG, CH), jnp.int32),
        pltpu.VMEM((2 * IG, CH), jnp.int32),
        pltpu.VMEM((NBUF, CH, DH), jnp.float32),
        [pltpu.SemaphoreType.DMA] * NBUF,
        pltpu.SemaphoreType.DMA,
        pltpu.SemaphoreType.DMA,
        pltpu.SemaphoreType.DMA,
        pltpu.SemaphoreType.DMA,
        pltpu.VMEM_SHARED((N_ACC, DH), jnp.float32),
    ],
    compiler_params=_sc_params,
)
def _sc_edges(src_hbm, dst_hbm, table_hbm, zeros_hbm, out_hbm,
              srcb, dstb, rows, semg, sems0, sems1, semi0, semi1, acc):
    cid = lax.axis_index("c")
    sid = lax.axis_index("s")
    wid = cid * NS + sid
    sbase = wid * NCHUNK
    dbase = sid * NCHUNK
    ssems = (sems0, sems1)

    def load_idx_group(g, slot, sem):
        pltpu.async_copy(
            src_hbm.at[pl.ds(sbase + g * IG, IG)],
            srcb.at[pl.ds(slot * IG, IG)], sem)
        pltpu.async_copy(
            dst_hbm.at[pl.ds(dbase + g * IG, IG)],
            dstb.at[pl.ds(slot * IG, IG)], sem)

    def wait_idx_group(slot, sem):
        pltpu.make_async_copy(
            src_hbm.at[pl.ds(0, IG)], srcb.at[pl.ds(slot * IG, IG)], sem
        ).wait()
        pltpu.make_async_copy(
            dst_hbm.at[pl.ds(0, IG)], dstb.at[pl.ds(slot * IG, IG)], sem
        ).wait()

    def start_gather(slot_row, buf):
        pltpu.async_copy(table_hbm.at[srcb.at[slot_row]],
                         rows.at[buf], semg[buf])

    def wait_gather(buf):
        pltpu.make_async_copy(
            table_hbm.at[pl.ds(0, CH)], rows.at[buf], semg[buf]).wait()

    def start_scatter(buf, slot_row, sem):
        pltpu.async_copy(rows.at[buf], acc.at[dstb.at[slot_row]], sem,
                         add=True)

    def wait_scatter(sem):
        pltpu.make_async_copy(
            zeros_hbm.at[pl.ds(0, CH)], rows.at[0], sem).wait()

    load_idx_group(0, 0, semi0)
    wait_idx_group(0, semi0)
    load_idx_group(1, 1, semi1)
    for k in range(3):
        start_gather(k, k)
    pltpu.sync_copy(zeros_hbm, acc.at[pl.ds(sid * NPT, NPT)])
    plsc.subcore_barrier()

    def pair(g2, _):
        base = g2 * 2 * IG
        for k in range(2 * IG):
            c = base + k
            buf = k % NBUF
            slot = k // IG
            row = k % IG
            wait_gather(buf)
            start_scatter(buf, slot * IG + row, ssems[k % 2])

            @pl.when(c >= 1)
            def _():
                wait_scatter(ssems[(k + 1) % 2])

            if k == 0:
                @pl.when(g2 >= 1)
                def _():
                    load_idx_group(2 * g2 + 1, 1, semi1)
            if k == IG:
                @pl.when(g2 < NG2 - 1)
                def _():
                    load_idx_group(2 * g2 + 2, 0, semi0)
            if k == 5:
                wait_idx_group(1, semi1)
            if k < 5:
                nslot_row = k + 3
            elif k < 13:
                nslot_row = IG + (k - 5)
            else:
                nslot_row = k - 13

            @pl.when(c + 3 < NCHUNK)
            def _():
                if k == 13:
                    wait_idx_group(0, semi0)
                start_gather(nslot_row, (k + 3) % NBUF)
        return _

    lax.fori_loop(0, NG2, pair, None)
    wait_scatter(ssems[(NCHUNK - 1) % 2])
    plsc.subcore_barrier()

    pltpu.sync_copy(acc.at[pl.ds(sid * NPT, NPT)], out_hbm.at[wid])


def _dinv_of(degp_blk):
    deg = jnp.sum(degp_blk, axis=1) + 1.0
    return lax.rsqrt(deg)


def _tc_prep_body(degp_ref, x_ref, w_ref, hh_ref):
    dinv = _dinv_of(degp_ref[...])
    h = jnp.dot(x_ref[...].astype(jnp.bfloat16),
                w_ref[...].astype(jnp.bfloat16),
                preferred_element_type=jnp.float32)
    hh = h * dinv[:, None]
    hh_ref[0] = hh[:, :DH]
    hh_ref[1] = hh[:, DH:]


def _tc_mid_body(agg_ref, hh_ref, degp_ref, b_ref, g_ref, be_ref, w_ref,
                 h1_ref, hh2_ref):
    dinv = _dinv_of(degp_ref[...])
    agg = jnp.concatenate([agg_ref[0], agg_ref[1]], axis=1)
    hh = jnp.concatenate([hh_ref[0], hh_ref[1]], axis=1)
    o = dinv[:, None] * (agg + hh) + b_ref[...]
    mu = jnp.mean(o, axis=-1, keepdims=True)
    var = jnp.mean((o - mu) ** 2, axis=-1, keepdims=True)
    o = (o - mu) * lax.rsqrt(var + 1e-5) * g_ref[...] + be_ref[...]
    h1 = jnp.maximum(o, 0.0)
    h1_ref[...] = h1
    u2 = jnp.dot(h1.astype(jnp.bfloat16),
                 w_ref[...].astype(jnp.bfloat16),
                 preferred_element_type=jnp.float32)
    hh2 = u2 * dinv[:, None]
    hh2_ref[0] = hh2[:, :DH]
    hh2_ref[1] = hh2[:, DH:]


def _tc_final_body(agg_ref, hh_ref, degp_ref, b_ref, g_ref, be_ref, h1_ref,
                   out_ref):
    dinv = _dinv_of(degp_ref[...])
    agg = jnp.concatenate([agg_ref[0], agg_ref[1]], axis=1)
    hh = jnp.concatenate([hh_ref[0], hh_ref[1]], axis=1)
    o = dinv[:, None] * (agg + hh) + b_ref[...]
    mu = jnp.mean(o, axis=-1, keepdims=True)
    var = jnp.mean((o - mu) ** 2, axis=-1, keepdims=True)
    o = (o - mu) * lax.rsqrt(var + 1e-5) * g_ref[...] + be_ref[...]
    out_ref[...] = h1_ref[...] + jnp.maximum(o, 0.0)


def _row_spec():
    return pl.BlockSpec((_BN, D), lambda i: (i, 0))


def _split_spec():
    return pl.BlockSpec((2, _BN, DH), lambda i: (0, i, 0))


def _degp_spec():
    return pl.BlockSpec((_BN, NW), lambda i: (i, 0))


def _vec_spec():
    return pl.BlockSpec((D,), lambda i: (0,))


_tc_prep = pl.pallas_call(
    _tc_prep_body,
    grid=(_GRID,),
    in_specs=[_degp_spec(), _row_spec(),
              pl.BlockSpec((D, D), lambda i: (0, 0))],
    out_specs=_split_spec(),
    out_shape=jax.ShapeDtypeStruct((2, N, DH), jnp.float32),
)

_tc_mid = pl.pallas_call(
    _tc_mid_body,
    grid=(_GRID,),
    in_specs=[_split_spec(), _split_spec(), _degp_spec(),
              _vec_spec(), _vec_spec(), _vec_spec(),
              pl.BlockSpec((D, D), lambda i: (0, 0))],
    out_specs=[_row_spec(), _split_spec()],
    out_shape=[
        jax.ShapeDtypeStruct((N, D), jnp.float32),
        jax.ShapeDtypeStruct((2, N, DH), jnp.float32),
    ],
)

_tc_final = pl.pallas_call(
    _tc_final_body,
    grid=(_GRID,),
    in_specs=[_split_spec(), _split_spec(), _degp_spec(),
              _vec_spec(), _vec_spec(), _vec_spec(), _row_spec()],
    out_specs=_row_spec(),
    out_shape=jax.ShapeDtypeStruct((N, D), jnp.float32),
)


@jax.jit
def kernel(x, edge_index, W1, b1, g1, be1, W2, b2, g2, be2):
    src = edge_index[0]
    dst = edge_index[1]
    pad = E_PAD - E
    pad_i = jnp.arange(pad, dtype=jnp.int32)
    src_p = jnp.concatenate([src, (pad_i * 41) % N])
    dst_p = jnp.concatenate([dst, N + (pad_i % 8)])

    sp = src_p.reshape(NS * NCHUNK, CH)
    src2 = jnp.concatenate([sp, sp + N], axis=0)
    dst2 = dst_p.reshape(NS * NCHUNK, CH)
    dstd = dst_p.reshape(NW, EPW)
    zrows = jnp.zeros((NPT, DH), jnp.float32)

    degp = _sc_deg(dstd).T

    hh1 = _tc_prep(degp, x, W1)
    agg1 = _sc_edges(src2, dst2, hh1.reshape(2 * N, DH), zrows)
    agg1 = agg1.reshape(2, N, DH)
    h1, hh2 = _tc_mid(agg1, hh1, degp, b1, g1, be1, W2)
    agg2 = _sc_edges(src2, dst2, hh2.reshape(2 * N, DH), zrows)
    agg2 = agg2.reshape(2, N, DH)
    return _tc_final(agg2, hh2, degp, b2, g2, be2, h1)


# speedup vs baseline: 1.0527x; 1.0527x over previous
import functools

import jax
import jax.numpy as jnp
from jax import lax
from jax.experimental import pallas as pl
from jax.experimental.pallas import tpu as pltpu
from jax.experimental.pallas import tpu_sc as plsc

N = 10000
D = 256
E = 160000
DH = D // 2

NC = 2
NS = 16
NW = NC * NS
L = 16

CH = 80
NCHUNK = 128
IG = 8
NG = NCHUNK // IG
NG2 = NG // 2
NBUF = 4
EPS = NCHUNK * CH
E_PAD = NS * EPS
EPW = E_PAD // NW
NPT = N // NS
N_ACC = N + 8
N_DEG = 10016

_BN = 5000
_GRID = N // _BN

_mesh = plsc.VectorSubcoreMesh(
    core_axis_name="c", subcore_axis_name="s", num_cores=NC, num_subcores=NS
)

_sc_params = pltpu.CompilerParams(needs_layout_passes=False)


@functools.partial(
    pl.kernel,
    out_type=jax.ShapeDtypeStruct((NW, N), jnp.float32),
    mesh=_mesh,
    scratch_types=[
        pltpu.VMEM((EPW,), jnp.int32),
        pltpu.VMEM((N,), jnp.float32),
    ],
    compiler_params=_sc_params,
)
def _sc_deg(dst_hbm, out_hbm, dstv, degp):
    cid = lax.axis_index("c")
    sid = lax.axis_index("s")
    wid = cid * NS + sid

    pltpu.sync_copy(dst_hbm.at[wid], dstv)

    zeros = jnp.zeros((L,), jnp.float32)

    def zbody(i, _):
        degp[pl.ds(i * L, L)] = zeros
        return _

    lax.fori_loop(0, N // L, zbody, None)

    ones = jnp.ones((L,), jnp.float32)

    nvec = jnp.full((L,), N, jnp.int32)

    def cbody(j, _):
        idx = dstv[pl.ds(j * L, L)]
        plsc.addupdate_scatter(degp, [idx], ones, mask=idx < nvec)
        return _

    lax.fori_loop(0, EPW // L, cbody, None)

    pltpu.sync_copy(degp, out_hbm.at[wid])


@functools.partial(
    pl.kernel,
    out_type=jax.ShapeDtypeStruct((NW, NPT, DH), jnp.float32),
    mesh=_mesh,
    scratch_types=[
        pltpu.VMEM((2 * IG, CH), jnp.int32),
        pltpu.VMEM((2 * IG, CH), jnp.int32),
        pltpu.VMEM((NBUF, CH, DH), jnp.float32),
        [pltpu.SemaphoreType.DMA] * NBUF,
        pltpu.SemaphoreType.DMA,
        pltpu.SemaphoreType.DMA,
        pltpu.SemaphoreType.DMA,
        pltpu.SemaphoreType.DMA,
        pltpu.VMEM_SHARED((N_ACC, DH), jnp.float32),
    ],
    compiler_params=_sc_params,
)
def _sc_edges(src_hbm, dst_hbm, table_hbm, zeros_hbm, out_hbm,
              srcb, dstb, rows, semg, sems0, sems1, semi0, semi1, acc):
    cid = lax.axis_index("c")
    sid = lax.axis_index("s")
    wid = cid * NS + sid
    sbase = wid * NCHUNK
    dbase = sid * NCHUNK
    ssems = (sems0, sems1)

    def load_idx_group(g, slot, sem):
        pltpu.async_copy(
            src_hbm.at[pl.ds(sbase + g * IG, IG)],
            srcb.at[pl.ds(slot * IG, IG)], sem)
        pltpu.async_copy(
            dst_hbm.at[pl.ds(dbase + g * IG, IG)],
            dstb.at[pl.ds(slot * IG, IG)], sem)

    def wait_idx_group(slot, sem):
        pltpu.make_async_copy(
            src_hbm.at[pl.ds(0, IG)], srcb.at[pl.ds(slot * IG, IG)], sem
        ).wait()
        pltpu.make_async_copy(
            dst_hbm.at[pl.ds(0, IG)], dstb.at[pl.ds(slot * IG, IG)], sem
        ).wait()

    def start_gather(slot_row, buf):
        pltpu.async_copy(table_hbm.at[srcb.at[slot_row]],
                         rows.at[buf], semg[buf])

    def wait_gather(buf):
        pltpu.make_async_copy(
            table_hbm.at[pl.ds(0, CH)], rows.at[buf], semg[buf]).wait()

    def start_scatter(buf, slot_row, sem):
        pltpu.async_copy(rows.at[buf], acc.at[dstb.at[slot_row]], sem,
                         add=True)

    def wait_scatter(sem):
        pltpu.make_async_copy(
            zeros_hbm.at[pl.ds(0, CH)], rows.at[0], sem).wait()

    load_idx_group(0, 0, semi0)
    wait_idx_group(0, semi0)
    load_idx_group(1, 1, semi1)
    for k in range(3):
        start_gather(k, k)
    pltpu.sync_copy(zeros_hbm, acc.at[pl.ds(sid * NPT, NPT)])
    plsc.subcore_barrier()

    def pair(g2, _):
        base = g2 * 2 * IG
        for k in range(2 * IG):
            c = base + k
            buf = k % NBUF
            slot = k // IG
            row = k % IG
            wait_gather(buf)
            start_scatter(buf, slot * IG + row, ssems[k % 2])

            @pl.when(c >= 1)
            def _():
                wait_scatter(ssems[(k + 1) % 2])

            if k == 0:
                @pl.when(g2 >= 1)
                def _():
                    load_idx_group(2 * g2 + 1, 1, semi1)
            if k == IG:
                @pl.when(g2 < NG2 - 1)
                def _():
                    load_idx_group(2 * g2 + 2, 0, semi0)
            if k == 5:
                wait_idx_group(1, semi1)
            if k < 5:
                nslot_row = k + 3
            elif k < 13:
                nslot_row = IG + (k - 5)
            else:
                nslot_row = k - 13

            @pl.when(c + 3 < NCHUNK)
            def _():
                if k == 13:
                    wait_idx_group(0, semi0)
                start_gather(nslot_row, (k + 3) % NBUF)
        return _

    lax.fori_loop(0, NG2, pair, None)
    wait_scatter(ssems[(NCHUNK - 1) % 2])
    plsc.subcore_barrier()

    pltpu.sync_copy(acc.at[pl.ds(sid * NPT, NPT)], out_hbm.at[wid])


def _dinv_of(degp_blk):
    deg = jnp.sum(degp_blk, axis=1) + 1.0
    return lax.rsqrt(deg)


def _tc_prep_body(degp_ref, x_ref, w_ref, hh_ref):
    dinv = _dinv_of(degp_ref[...])
    h = jnp.dot(x_ref[...], w_ref[...], preferred_element_type=jnp.float32)
    hh = h * dinv[:, None]
    hh_ref[0] = hh[:, :DH]
    hh_ref[1] = hh[:, DH:]


def _agg_of(agg_ref):
    a = agg_ref[...]
    a = a.reshape(2, _BN, DH)
    return jnp.concatenate([a[0], a[1]], axis=1)


def _tc_mid_body(agg_ref, hh_ref, degp_ref, b_ref, g_ref, be_ref, w_ref,
                 h1_ref, hh2_ref):
    dinv = _dinv_of(degp_ref[...])
    agg = _agg_of(agg_ref)
    hh = jnp.concatenate([hh_ref[0], hh_ref[1]], axis=1)
    o = dinv[:, None] * (agg + hh) + b_ref[...]
    mu = jnp.mean(o, axis=-1, keepdims=True)
    var = jnp.mean((o - mu) ** 2, axis=-1, keepdims=True)
    o = (o - mu) * lax.rsqrt(var + 1e-5) * g_ref[...] + be_ref[...]
    h1 = jnp.maximum(o, 0.0)
    h1_ref[...] = h1
    u2 = jnp.dot(h1, w_ref[...], preferred_element_type=jnp.float32)
    hh2 = u2 * dinv[:, None]
    hh2_ref[0] = hh2[:, :DH]
    hh2_ref[1] = hh2[:, DH:]


def _tc_final_body(agg_ref, hh_ref, degp_ref, b_ref, g_ref, be_ref, h1_ref,
                   out_ref):
    dinv = _dinv_of(degp_ref[...])
    agg = _agg_of(agg_ref)
    hh = jnp.concatenate([hh_ref[0], hh_ref[1]], axis=1)
    o = dinv[:, None] * (agg + hh) + b_ref[...]
    mu = jnp.mean(o, axis=-1, keepdims=True)
    var = jnp.mean((o - mu) ** 2, axis=-1, keepdims=True)
    o = (o - mu) * lax.rsqrt(var + 1e-5) * g_ref[...] + be_ref[...]
    out_ref[...] = h1_ref[...] + jnp.maximum(o, 0.0)


def _row_spec():
    return pl.BlockSpec((_BN, D), lambda i: (i, 0))


def _split_spec():
    return pl.BlockSpec((2, _BN, DH), lambda i: (0, i, 0))


def _degp_spec():
    return pl.BlockSpec((_BN, NW), lambda i: (i, 0))


def _agg_spec():
    return pl.BlockSpec((NC, _BN // NPT, NPT, DH), lambda i: (0, i, 0, 0))


def _vec_spec():
    return pl.BlockSpec((D,), lambda i: (0,))


_tc_prep = pl.pallas_call(
    _tc_prep_body,
    grid=(_GRID,),
    in_specs=[_degp_spec(), _row_spec(),
              pl.BlockSpec((D, D), lambda i: (0, 0))],
    out_specs=_split_spec(),
    out_shape=jax.ShapeDtypeStruct((2, N, DH), jnp.float32),
)

_tc_mid = pl.pallas_call(
    _tc_mid_body,
    grid=(_GRID,),
    in_specs=[_agg_spec(), _split_spec(), _degp_spec(),
              _vec_spec(), _vec_spec(), _vec_spec(),
              pl.BlockSpec((D, D), lambda i: (0, 0))],
    out_specs=[_row_spec(), _split_spec()],
    out_shape=[
        jax.ShapeDtypeStruct((N, D), jnp.float32),
        jax.ShapeDtypeStruct((2, N, DH), jnp.float32),
    ],
)

_tc_final = pl.pallas_call(
    _tc_final_body,
    grid=(_GRID,),
    in_specs=[_agg_spec(), _split_spec(), _degp_spec(),
              _vec_spec(), _vec_spec(), _vec_spec(), _row_spec()],
    out_specs=_row_spec(),
    out_shape=jax.ShapeDtypeStruct((N, D), jnp.float32),
)


@jax.jit
def kernel(x, edge_index, W1, b1, g1, be1, W2, b2, g2, be2):
    src = edge_index[0]
    dst = edge_index[1]
    pad = E_PAD - E
    pad_i = jnp.arange(pad, dtype=jnp.int32)
    src_p = jnp.concatenate([src, (pad_i * 41) % N])
    dst_p = jnp.concatenate([dst, N + (pad_i % 8)])

    sp = src_p.reshape(NS * NCHUNK, CH)
    src2 = jnp.concatenate([sp, sp + N], axis=0)
    dst2 = dst_p.reshape(NS * NCHUNK, CH)
    dstd = dst_p.reshape(NW, EPW)
    zrows = jnp.zeros((NPT, DH), jnp.float32)

    degp = _sc_deg(dstd).T

    hh1 = _tc_prep(degp, x, W1)
    agg1 = _sc_edges(src2, dst2, hh1.reshape(2 * N, DH), zrows)
    agg1 = agg1.reshape(NC, NS, NPT, DH)
    h1, hh2 = _tc_mid(agg1, hh1, degp, b1, g1, be1, W2)
    agg2 = _sc_edges(src2, dst2, hh2.reshape(2 * N, DH), zrows)
    agg2 = agg2.reshape(NC, NS, NPT, DH)
    return _tc_final(agg2, hh2, degp, b2, g2, be2, h1)
